# baseline (speedup 1.0000x reference)
import functools

import jax
import jax.numpy as jnp
from jax import lax
from jax.experimental import pallas as pl
from jax.experimental.pallas import tpu as pltpu
from jax.experimental.pallas import tpu_sc as plsc

B, HW, K, DIN, DOUT = 4, 4096, 16, 2048, 2048
NC, NS, L = 2, 16, 16
NW = NC * NS
SEG = (B * HW) // NW
CH = 16
NCH = SEG // CH
WPB = NW // B


def _mm_body(c_ref, w_ref, y_ref):
    y_ref[...] = lax.dot_general(
        c_ref[...], w_ref[...],
        dimension_numbers=(((1,), (1,)), ((), ())),
        preferred_element_type=jnp.float32,
    )


def _codes_matmul(codes_flat, W):
    n_blk = 4
    return pl.pallas_call(
        _mm_body,
        grid=(n_blk,),
        in_specs=[
            pl.BlockSpec((B * K, DIN), lambda j: (0, 0)),
            pl.BlockSpec((DOUT // n_blk, DIN), lambda j: (j, 0)),
        ],
        out_specs=pl.BlockSpec((B * K, DOUT // n_blk), lambda j: (0, j)),
        out_shape=jax.ShapeDtypeStruct((B * K, DOUT), jnp.float32),
    )(codes_flat, W)


def _sc_body(y_hbm, masks_hbm, bfc_hbm, scale_hbm, bias_hbm, out_hbm,
             m_l, y_l, bfc_l, sc_l, bi_l, stage0, stage1, sem0, sem1):
    wid = lax.axis_index("s") * NC + lax.axis_index("c")
    b_idx = wid // WPB
    n0 = (wid % WPB) * SEG

    pltpu.sync_copy(y_hbm.at[b_idx], y_l)
    pltpu.sync_copy(masks_hbm.at[b_idx, :, pl.ds(n0, SEG)], m_l)
    pltpu.sync_copy(bfc_hbm, bfc_l)
    pltpu.sync_copy(scale_hbm, sc_l)
    pltpu.sync_copy(bias_hbm, bi_l)
    scale = sc_l[...][0]
    bias = bi_l[...][0]

    def compute_chunk(c, stage):
        t0 = c * CH
        m = m_l[0, pl.ds(t0, CH)]
        a = jnp.zeros((CH,), jnp.int32)
        for kk in range(1, K):
            x = m_l[kk, pl.ds(t0, CH)]
            upd = x > m
            a = jnp.where(upd, kk, a)
            m = jnp.where(upd, x, m)
        g = 1.0 / (1.0 + jnp.exp(-(scale * m + bias)))
        a_ts = [a[t] for t in range(CH)]
        g_vs = [jnp.full((L,), g[t]) for t in range(CH)]

        @plsc.parallel_loop(0, DOUT, step=L, unroll=16)
        def i_body(o):
            bias_vec = bfc_l[pl.ds(o, L)]
            for t in range(CH):
                stage[t, pl.ds(o, L)] = g_vs[t] * y_l[a_ts[t], pl.ds(o, L)] + bias_vec

    def out_slice(c):
        return out_hbm.at[b_idx, pl.ds(n0 + c * CH, CH)]

    def pair_body(p, _):
        @pl.when(p > 0)
        def _w0():
            pltpu.make_async_copy(stage0, out_slice(0), sem0).wait()
        compute_chunk(2 * p, stage0)
        pltpu.async_copy(stage0, out_slice(2 * p), sem0)

        @pl.when(p > 0)
        def _w1():
            pltpu.make_async_copy(stage1, out_slice(0), sem1).wait()
        compute_chunk(2 * p + 1, stage1)
        pltpu.async_copy(stage1, out_slice(2 * p + 1), sem1)
        return 0

    lax.fori_loop(0, NCH // 2, pair_body, 0)
    pltpu.make_async_copy(stage0, out_slice(0), sem0).wait()
    pltpu.make_async_copy(stage1, out_slice(0), sem1).wait()


@functools.lru_cache(maxsize=1)
def _make_sc_dispatch():
  return functools.partial(
    pl.kernel,
    out_type=jax.ShapeDtypeStruct((B, HW, DOUT), jnp.float32),
    mesh=plsc.VectorSubcoreMesh(
        core_axis_name="c", subcore_axis_name="s",
        num_cores=NC, num_subcores=NS),
    scratch_types=[
        pltpu.VMEM((K, SEG), jnp.float32),
        pltpu.VMEM((K, DOUT), jnp.float32),
        pltpu.VMEM((DOUT,), jnp.float32),
        pltpu.VMEM((L,), jnp.float32),
        pltpu.VMEM((L,), jnp.float32),
        pltpu.VMEM((CH, DOUT), jnp.float32),
        pltpu.VMEM((CH, DOUT), jnp.float32),
        pltpu.SemaphoreType.DMA,
        pltpu.SemaphoreType.DMA,
    ],
  )(_sc_body)


def kernel(inputs, codes, masks, W, b_fc, scale, bias):
    del inputs
    y = _codes_matmul(codes.reshape(B * K, DIN), W).reshape(B, K, DOUT)
    scale_v = jnp.broadcast_to(scale.astype(jnp.float32), (L,))
    bias_v = jnp.broadcast_to(bias.astype(jnp.float32), (L,))
    return _make_sc_dispatch()(y, masks, b_fc, scale_v, bias_v)


# speedup vs baseline: 1.3663x; 1.3663x over previous
import functools

import jax
import jax.numpy as jnp
from jax import lax
from jax.experimental import pallas as pl
from jax.experimental.pallas import tpu as pltpu
from jax.experimental.pallas import tpu_sc as plsc

B, HW, K, DIN, DOUT = 4, 4096, 16, 2048, 2048
NC, NS, L = 2, 16, 16
NW = NC * NS
SEG = (B * HW) // NW
CH = 16
NCH = SEG // CH
WPB = NW // B


def _mm_body(c_ref, w_ref, y_ref):
    y_ref[...] = lax.dot_general(
        c_ref[...], w_ref[...],
        dimension_numbers=(((1,), (1,)), ((), ())),
        preferred_element_type=jnp.float32,
    )


def _codes_matmul(codes_flat, W):
    n_blk = 4
    return pl.pallas_call(
        _mm_body,
        grid=(n_blk,),
        in_specs=[
            pl.BlockSpec((B * K, DIN), lambda j: (0, 0)),
            pl.BlockSpec((DOUT // n_blk, DIN), lambda j: (j, 0)),
        ],
        out_specs=pl.BlockSpec((B * K, DOUT // n_blk), lambda j: (0, j)),
        out_shape=jax.ShapeDtypeStruct((B * K, DOUT), jnp.float32),
    )(codes_flat, W)


def _sc_body(y_hbm, masks_hbm, aux_hbm, out_hbm,
             m_l, y_l, aux_l, stage0, stage1, sem0, sem1):
    wid = lax.axis_index("s") * NC + lax.axis_index("c")
    b_idx = wid // WPB
    n0 = (wid % WPB) * SEG

    cp_y = pltpu.async_copy(y_hbm.at[b_idx], y_l, sem0)
    cp_m = pltpu.async_copy(masks_hbm.at[b_idx, :, pl.ds(n0, SEG)], m_l, sem1)
    pltpu.sync_copy(aux_hbm, aux_l)
    cp_y.wait()
    cp_m.wait()
    bfc_l = aux_l.at[pl.ds(0, DOUT)]
    scale = aux_l[pl.ds(DOUT, L)][0]
    bias = aux_l[pl.ds(DOUT + L, L)][0]

    def compute_chunk(c, stage):
        t0 = c * CH
        m = m_l[0, pl.ds(t0, CH)]
        a = jnp.zeros((CH,), jnp.int32)
        for kk in range(1, K):
            x = m_l[kk, pl.ds(t0, CH)]
            upd = x > m
            a = jnp.where(upd, kk, a)
            m = jnp.where(upd, x, m)
        g = 1.0 / (1.0 + jnp.exp(-(scale * m + bias)))
        a_ts = [a[t] for t in range(CH)]
        g_vs = [jnp.full((L,), g[t]) for t in range(CH)]

        @plsc.parallel_loop(0, DOUT, step=L, unroll=8)
        def i_body(o):
            bias_vec = bfc_l[pl.ds(o, L)]
            for t in range(CH):
                stage[t, pl.ds(o, L)] = g_vs[t] * y_l[a_ts[t], pl.ds(o, L)] + bias_vec

    def out_slice(c):
        return out_hbm.at[b_idx, pl.ds(n0 + c * CH, CH)]

    def pair_body(p, _):
        @pl.when(p > 0)
        def _w0():
            pltpu.make_async_copy(stage0, out_slice(0), sem0).wait()
        compute_chunk(2 * p, stage0)
        pltpu.async_copy(stage0, out_slice(2 * p), sem0)

        @pl.when(p > 0)
        def _w1():
            pltpu.make_async_copy(stage1, out_slice(0), sem1).wait()
        compute_chunk(2 * p + 1, stage1)
        pltpu.async_copy(stage1, out_slice(2 * p + 1), sem1)
        return 0

    lax.fori_loop(0, NCH // 2, pair_body, 0)
    pltpu.make_async_copy(stage0, out_slice(0), sem0).wait()
    pltpu.make_async_copy(stage1, out_slice(0), sem1).wait()


@functools.lru_cache(maxsize=1)
def _make_sc_dispatch():
  return functools.partial(
    pl.kernel,
    out_type=jax.ShapeDtypeStruct((B, HW, DOUT), jnp.float32),
    mesh=plsc.VectorSubcoreMesh(
        core_axis_name="c", subcore_axis_name="s",
        num_cores=NC, num_subcores=NS),
    scratch_types=[
        pltpu.VMEM((K, SEG), jnp.float32),
        pltpu.VMEM((K, DOUT), jnp.float32),
        pltpu.VMEM((DOUT + 2 * L,), jnp.float32),
        pltpu.VMEM((CH, DOUT), jnp.float32),
        pltpu.VMEM((CH, DOUT), jnp.float32),
        pltpu.SemaphoreType.DMA,
        pltpu.SemaphoreType.DMA,
    ],
  )(_sc_body)


def kernel(inputs, codes, masks, W, b_fc, scale, bias):
    del inputs
    y = _codes_matmul(codes.reshape(B * K, DIN), W).reshape(B, K, DOUT)
    aux = jnp.concatenate([
        b_fc.astype(jnp.float32),
        jnp.broadcast_to(scale.astype(jnp.float32), (L,)),
        jnp.broadcast_to(bias.astype(jnp.float32), (L,)),
    ])
    return _make_sc_dispatch()(y, masks, aux)
